# Initial kernel scaffold; baseline (speedup 1.0000x reference)
import jax
import jax.numpy as jnp
from jax.experimental import pallas as pl


def kernel(vertex_features, edge_features, edge_index, eW1, eb1, eW2, eb2, vW1, vb1, vW2, vb2):
    raise NotImplementedError("write your pallas kernel here")


# speedup vs baseline: 2.8007x; 2.8007x over previous
import functools

import jax
import jax.numpy as jnp
from jax import lax
from jax.experimental import pallas as pl
from jax.experimental.pallas import tpu as pltpu
from jax.experimental.pallas import tpu_sc as plsc

N_NODES = 10000
N_EDGES = 320000
H = 128

NC = 2
NS = 16
NW = NC * NS
EPW = N_EDGES // NW
SUB = 80
NB = 5
GRP = SUB * NB
NGRP = EPW // GRP

SUB_S = 40
NB_S = 5
GRP_S = SUB_S * NB_S
NGRP_S = EPW // GRP_S
NROWCH = N_NODES // GRP_S


def _premix_body(v_ref, ws_ref, wr_ref, ps_ref, pr_ref):
    v = v_ref[...]
    ps_ref[...] = jnp.dot(v, ws_ref[...], preferred_element_type=jnp.float32)
    pr_ref[...] = jnp.dot(v, wr_ref[...], preferred_element_type=jnp.float32)


def _premix(v, ws, wr):
    return pl.pallas_call(
        _premix_body,
        out_shape=(
            jax.ShapeDtypeStruct((N_NODES, H), jnp.float32),
            jax.ShapeDtypeStruct((N_NODES, H), jnp.float32),
        ),
    )(v, ws, wr)


def _edge_body(g_ref, e_ref, we_ref, b1_ref, w2_ref, b2_ref, o_ref):
    pre = (g_ref[...]
           + jnp.dot(e_ref[...], we_ref[...], preferred_element_type=jnp.float32)
           + b1_ref[...])
    h = jnp.maximum(pre, 0.0)
    o_ref[...] = jnp.dot(h, w2_ref[...], preferred_element_type=jnp.float32) + b2_ref[...]


def _edge_mlp(g, e, we, b1, w2, b2):
    bm = 512
    grid = (N_EDGES // bm,)
    return pl.pallas_call(
        _edge_body,
        grid=grid,
        in_specs=[
            pl.BlockSpec((bm, H), lambda i: (i, 0)),
            pl.BlockSpec((bm, H), lambda i: (i, 0)),
            pl.BlockSpec((H, H), lambda i: (0, 0)),
            pl.BlockSpec((1, H), lambda i: (0, 0)),
            pl.BlockSpec((H, H), lambda i: (0, 0)),
            pl.BlockSpec((1, H), lambda i: (0, 0)),
        ],
        out_specs=pl.BlockSpec((bm, H), lambda i: (i, 0)),
        out_shape=jax.ShapeDtypeStruct((N_EDGES, H), jnp.float32),
    )(g, e, we, b1.reshape(1, H), w2, b2.reshape(1, H))


def _vertex_body(v_ref, p_ref, wv_ref, wa_ref, b1_ref, w2_ref, b2_ref, o_ref):
    aggr = p_ref[0] + p_ref[1]
    pre = (jnp.dot(v_ref[...], wv_ref[...], preferred_element_type=jnp.float32)
           + jnp.dot(aggr, wa_ref[...], preferred_element_type=jnp.float32)
           + b1_ref[...])
    h = jnp.maximum(pre, 0.0)
    o_ref[...] = jnp.dot(h, w2_ref[...], preferred_element_type=jnp.float32) + b2_ref[...]


def _vertex_mlp(v, partials, wv, wa, b1, w2, b2):
    return pl.pallas_call(
        _vertex_body,
        out_shape=jax.ShapeDtypeStruct((N_NODES, H), jnp.float32),
    )(v, partials, wv, wa, b1.reshape(1, H), w2, b2.reshape(1, H))


def _gather_add(ps, pr, sidx3, ridx3):
    mesh = plsc.VectorSubcoreMesh(core_axis_name="c", subcore_axis_name="s")

    @functools.partial(
        pl.kernel,
        out_type=jax.ShapeDtypeStruct((N_EDGES, H), jnp.float32),
        mesh=mesh,
        scratch_types=[
            pltpu.VMEM((NB, SUB), jnp.int32),
            pltpu.VMEM((NB, SUB), jnp.int32),
            pltpu.VMEM((GRP, H), jnp.float32),
            pltpu.VMEM((GRP, H), jnp.float32),
            pltpu.SemaphoreType.DMA,
        ],
    )
    def k(ps_hbm, pr_hbm, s_hbm, r_hbm, out_hbm, si_v, ri_v, bs_v, br_v, sem):
        wid = lax.axis_index("s") * NC + lax.axis_index("c")
        row0 = wid * NGRP

        def body(g, _):
            grow = row0 + g
            pltpu.sync_copy(s_hbm.at[grow], si_v)
            pltpu.sync_copy(r_hbm.at[grow], ri_v)
            descs = []
            for j in range(NB):
                descs.append(pltpu.async_copy(
                    ps_hbm.at[si_v.at[j]], bs_v.at[pl.ds(j * SUB, SUB)], sem))
                descs.append(pltpu.async_copy(
                    pr_hbm.at[ri_v.at[j]], br_v.at[pl.ds(j * SUB, SUB)], sem))
            for d in descs:
                d.wait()

            def addb(e, _):
                for cc in range(H // 16):
                    sl = pl.ds(cc * 16, 16)
                    bs_v[e, sl] = bs_v[e, sl] + br_v[e, sl]
                return 0

            lax.fori_loop(0, GRP, addb, 0)
            pltpu.sync_copy(bs_v, out_hbm.at[pl.ds(grow * GRP, GRP)])
            return 0

        lax.fori_loop(0, NGRP, body, 0)

    return k(ps, pr, sidx3, ridx3)


def _scatter_add(newe, ridx3):
    mesh = plsc.VectorSubcoreMesh(core_axis_name="c", subcore_axis_name="s")

    @functools.partial(
        pl.kernel,
        out_type=jax.ShapeDtypeStruct((NC * N_NODES, H), jnp.float32),
        mesh=mesh,
        scratch_types=[
            pltpu.VMEM((NB_S, SUB_S), jnp.int32),
            pltpu.VMEM((GRP_S, H), jnp.float32),
            pltpu.VMEM_SHARED((N_NODES, H), jnp.float32),
        ],
    )
    def k(e_hbm, r_hbm, out_hbm, ri_v, buf_v, acc_sh):
        cid = lax.axis_index("c")
        sid = lax.axis_index("s")
        wid = sid * NC + cid

        def zb(e, _):
            for cc in range(H // 16):
                buf_v[e, pl.ds(cc * 16, 16)] = jnp.zeros((16,), jnp.float32)
            return 0

        lax.fori_loop(0, GRP_S, zb, 0)
        for j in range(4):
            ch = sid + NS * j

            @pl.when(ch < NROWCH)
            def _():
                pltpu.sync_copy(buf_v, acc_sh.at[pl.ds(ch * GRP_S, GRP_S)])

        plsc.subcore_barrier()

        def body(g, _):
            grow = wid * NGRP_S + g
            pltpu.sync_copy(r_hbm.at[grow], ri_v)
            pltpu.sync_copy(e_hbm.at[pl.ds(grow * GRP_S, GRP_S)], buf_v)
            for j in range(NB_S):
                pltpu.sync_copy(buf_v.at[pl.ds(j * SUB_S, SUB_S)],
                                acc_sh.at[ri_v.at[j]], add=True)
            return 0

        lax.fori_loop(0, NGRP_S, body, 0)
        plsc.subcore_barrier()

        for j in range(4):
            ch = sid + NS * j

            @pl.when(ch < NROWCH)
            def _():
                pltpu.sync_copy(acc_sh.at[pl.ds(ch * GRP_S, GRP_S)],
                                out_hbm.at[pl.ds(cid * N_NODES + ch * GRP_S, GRP_S)])

    return k(newe, ridx3)


def kernel(vertex_features, edge_features, edge_index, eW1, eb1, eW2, eb2,
           vW1, vb1, vW2, vb2):
    senders = edge_index[0].astype(jnp.int32)
    receivers = edge_index[1].astype(jnp.int32)
    sidx3 = senders.reshape(NW * NGRP, NB, SUB)
    ridx3 = receivers.reshape(NW * NGRP, NB, SUB)
    ridx3s = receivers.reshape(NW * NGRP_S, NB_S, SUB_S)

    ws, wr, we = eW1[:H], eW1[H:2 * H], eW1[2 * H:]
    ps, pr = _premix(vertex_features, ws, wr)
    g = _gather_add(ps, pr, sidx3, ridx3)
    new_edge = _edge_mlp(g, edge_features, we, eb1, eW2, eb2)
    partials = _scatter_add(new_edge, ridx3s)
    partials = partials.reshape(NC, N_NODES, H)
    new_vertex = _vertex_mlp(vertex_features, partials, vW1[:H], vW1[H:],
                             vb1, vW2, vb2)
    return (new_vertex, new_edge)
